# baseline (speedup 1.0000x reference)
import jax
import jax.numpy as jnp
from jax import lax
from jax.experimental import pallas as pl
from jax.experimental.pallas import tpu as pltpu
from jax.experimental.pallas import tpu_sc as plsc

VOCAB = 100000
SEQ_LEN = 50
EMBED = 64
BATCH = 4096

_NC = 2
_NS = 16
_NW = _NC * _NS

_B_TOTAL = BATCH * SEQ_LEN
_B_PER_W = _B_TOTAL // _NW
_CHUNK = 128
_N_CHUNKS = _B_PER_W // _CHUNK
_K = 5
_BIG = _K * _CHUNK
_N_BIG = _N_CHUNKS // _K


def _emb_kernel(x_hbm, table_hbm, out_hbm, idx_v, rows_v,
                gsem0, gsem1, ssem0, ssem1):
    wid = lax.axis_index("s") * _NC + lax.axis_index("c")
    base = wid * _B_PER_W
    pltpu.sync_copy(x_hbm.at[wid], idx_v)

    gsems = (gsem0, gsem1)
    ssems = (ssem0, ssem1)

    def fire(j, b):
        for t in range(_K):
            pltpu.async_copy(
                table_hbm.at[idx_v.at[j * _K + t]],
                rows_v.at[b, pl.ds(t * _CHUNK, _CHUNK)],
                gsems[b],
            )

    def drain(j, b):
        for t in range(_K):
            pltpu.make_async_copy(
                table_hbm.at[idx_v.at[j * _K + t]],
                rows_v.at[b, pl.ds(t * _CHUNK, _CHUNK)],
                gsems[b],
            ).wait()

    def store_fire(j, b):
        pltpu.async_copy(
            rows_v.at[b], out_hbm.at[pl.ds(base + j * _BIG, _BIG)], ssems[b]
        )

    def store_wait(j, b):
        pltpu.make_async_copy(
            rows_v.at[b], out_hbm.at[pl.ds(base + j * _BIG, _BIG)], ssems[b]
        ).wait()

    fire(0, 0)
    for i in range(_N_BIG):
        b = i & 1
        drain(i, b)
        if i + 1 < _N_BIG:
            if i >= 1:
                store_wait(i - 1, 1 - b)
            fire(i + 1, 1 - b)
        store_fire(i, b)
    store_wait(_N_BIG - 2, (_N_BIG - 2) & 1)
    store_wait(_N_BIG - 1, (_N_BIG - 1) & 1)


@jax.jit
def kernel(x, table):
    x_flat = x.reshape(_NW, _N_CHUNKS, _CHUNK).astype(jnp.int32)
    mesh = plsc.VectorSubcoreMesh(core_axis_name="c", subcore_axis_name="s")
    out = pl.kernel(
        _emb_kernel,
        mesh=mesh,
        out_type=jax.ShapeDtypeStruct((_B_TOTAL, EMBED), jnp.float32),
        scratch_types=[
            pltpu.VMEM((_N_CHUNKS, _CHUNK), jnp.int32),
            pltpu.VMEM((2, _BIG, EMBED), jnp.float32),
            pltpu.SemaphoreType.DMA,
            pltpu.SemaphoreType.DMA,
            pltpu.SemaphoreType.DMA,
            pltpu.SemaphoreType.DMA,
        ],
        compiler_params=pltpu.CompilerParams(use_tc_tiling_on_sc=False),
    )(x_flat, table)
    return out.reshape(BATCH, SEQ_LEN, EMBED)


# speedup vs baseline: 1.0098x; 1.0098x over previous
import jax
import jax.numpy as jnp
from jax import lax
from jax.experimental import pallas as pl
from jax.experimental.pallas import tpu as pltpu
from jax.experimental.pallas import tpu_sc as plsc

VOCAB = 100000
SEQ_LEN = 50
EMBED = 64
BATCH = 4096

_NC = 2
_NS = 16
_NW = _NC * _NS

_B_TOTAL = BATCH * SEQ_LEN
_B_PER_W = _B_TOTAL // _NW
_CHUNK = 128
_N_CHUNKS = _B_PER_W // _CHUNK
_K = 5
_BIG = _K * _CHUNK
_N_BIG = _N_CHUNKS // _K


def _emb_kernel(x_hbm, table_hbm, out_hbm, idx_v, rows_v,
                gsem0, gsem1, ssem0, ssem1):
    wid = lax.axis_index("s") * _NC + lax.axis_index("c")
    base = wid * _B_PER_W
    pltpu.sync_copy(x_hbm.at[wid], idx_v)

    gsems = (gsem0, gsem1)
    ssems = (ssem0, ssem1)

    def fire(j, b):
        for t in range(_K):
            pltpu.async_copy(
                table_hbm.at[idx_v.at[j * _K + t]],
                rows_v.at[b, pl.ds(t * _CHUNK, _CHUNK)],
                gsems[b],
            )

    def drain(j, b):
        for t in range(_K):
            pltpu.make_async_copy(
                table_hbm.at[idx_v.at[j * _K + t]],
                rows_v.at[b, pl.ds(t * _CHUNK, _CHUNK)],
                gsems[b],
            ).wait()

    def store_fire(j, b):
        pltpu.async_copy(
            rows_v.at[b], out_hbm.at[pl.ds(base + j * _BIG, _BIG)], ssems[b]
        )

    def store_wait(j, b):
        pltpu.make_async_copy(
            rows_v.at[b], out_hbm.at[pl.ds(base + j * _BIG, _BIG)], ssems[b]
        ).wait()

    fire(0, 0)
    for i in range(_N_BIG):
        b = i & 1
        if i + 1 < _N_BIG:
            if i >= 1:
                store_wait(i - 1, 1 - b)
            fire(i + 1, 1 - b)
        drain(i, b)
        store_fire(i, b)
    store_wait(_N_BIG - 2, (_N_BIG - 2) & 1)
    store_wait(_N_BIG - 1, (_N_BIG - 1) & 1)


@jax.jit
def kernel(x, table):
    x_flat = x.reshape(_NW, _N_CHUNKS, _CHUNK).astype(jnp.int32)
    mesh = plsc.VectorSubcoreMesh(core_axis_name="c", subcore_axis_name="s")
    out = pl.kernel(
        _emb_kernel,
        mesh=mesh,
        out_type=jax.ShapeDtypeStruct((_B_TOTAL, EMBED), jnp.float32),
        scratch_types=[
            pltpu.VMEM((_N_CHUNKS, _CHUNK), jnp.int32),
            pltpu.VMEM((2, _BIG, EMBED), jnp.float32),
            pltpu.SemaphoreType.DMA,
            pltpu.SemaphoreType.DMA,
            pltpu.SemaphoreType.DMA,
            pltpu.SemaphoreType.DMA,
        ],
        compiler_params=pltpu.CompilerParams(use_tc_tiling_on_sc=False),
    )(x_flat, table)
    return out.reshape(BATCH, SEQ_LEN, EMBED)


# speedup vs baseline: 1.0650x; 1.0547x over previous
import jax
import jax.numpy as jnp
from jax import lax
from jax.experimental import pallas as pl
from jax.experimental.pallas import tpu as pltpu
from jax.experimental.pallas import tpu_sc as plsc

VOCAB = 100000
SEQ_LEN = 50
EMBED = 64
BATCH = 4096

_NC = 2
_NS = 16
_NW = _NC * _NS

_B_TOTAL = BATCH * SEQ_LEN
_B_PER_W = _B_TOTAL // _NW
_CHUNK = 128
_N_CHUNKS = _B_PER_W // _CHUNK
_K = 5
_BIG = _K * _CHUNK
_N_BIG = _N_CHUNKS // _K


def _emb_kernel(x_hbm, table_hbm, out_hbm, idx_v, rows_v,
                gsem0, gsem1, ssem0, ssem1):
    wid = lax.axis_index("s") * _NC + lax.axis_index("c")
    base = wid * _B_PER_W
    pltpu.sync_copy(x_hbm.at[wid], idx_v)

    gsems = (gsem0, gsem1)
    ssems = (ssem0, ssem1)

    def fire(j, b):
        for t in range(_K):
            pltpu.async_copy(
                table_hbm.at[idx_v.at[j * _K + t]],
                rows_v.at[b, pl.ds(t * _CHUNK, _CHUNK)],
                gsems[b],
            )

    def drain(j, b):
        for t in range(_K):
            pltpu.make_async_copy(
                table_hbm.at[idx_v.at[j * _K + t]],
                rows_v.at[b, pl.ds(t * _CHUNK, _CHUNK)],
                gsems[b],
            ).wait()

    def store_fire(j, b):
        pltpu.async_copy(
            rows_v.at[b], out_hbm.at[pl.ds(base + j * _BIG, _BIG)], ssems[b]
        )

    def store_wait(j, b):
        pltpu.make_async_copy(
            rows_v.at[b], out_hbm.at[pl.ds(base + j * _BIG, _BIG)], ssems[b]
        ).wait()

    fire(0, 0)
    for i in range(_N_BIG):
        b = i & 1
        if i + 1 < _N_BIG:
            if i >= 1:
                store_wait(i - 1, 1 - b)
            fire(i + 1, 1 - b)
        drain(i, b)
        store_fire(i, b)
    store_wait(_N_BIG - 2, (_N_BIG - 2) & 1)
    store_wait(_N_BIG - 1, (_N_BIG - 1) & 1)


@jax.jit
def kernel(x, table):
    x_flat = x.T.reshape(_NW, _N_CHUNKS, _CHUNK).astype(jnp.int32)
    mesh = plsc.VectorSubcoreMesh(core_axis_name="c", subcore_axis_name="s")
    out = pl.kernel(
        _emb_kernel,
        mesh=mesh,
        out_type=jax.ShapeDtypeStruct((_B_TOTAL, EMBED), jnp.float32),
        scratch_types=[
            pltpu.VMEM((_N_CHUNKS, _CHUNK), jnp.int32),
            pltpu.VMEM((2, _BIG, EMBED), jnp.float32),
            pltpu.SemaphoreType.DMA,
            pltpu.SemaphoreType.DMA,
            pltpu.SemaphoreType.DMA,
            pltpu.SemaphoreType.DMA,
        ],
        compiler_params=pltpu.CompilerParams(use_tc_tiling_on_sc=False),
    )(x_flat, table)
    return out.reshape(SEQ_LEN, BATCH, EMBED).transpose(1, 0, 2)
